# baseline (speedup 1.0000x reference)
import functools

import jax
import jax.numpy as jnp
from jax import lax
from jax.experimental import pallas as pl
from jax.experimental.pallas import tpu as pltpu
from jax.experimental.pallas import tpu_sc as plsc

B, T = 16384, 200
NC, NS, L = 2, 16, 16
NW = NC * NS
RPW = B // NW
CHUNK = 256
NCHUNK = RPW // CHUNK
GROUPS = CHUNK // L


@functools.partial(
    pl.kernel,
    out_type=jax.ShapeDtypeStruct((B, T), jnp.float32),
    mesh=plsc.VectorSubcoreMesh(core_axis_name="c", subcore_axis_name="s"),
    scratch_types=[pltpu.VMEM((CHUNK, T), jnp.float32)],
    compiler_params=pltpu.CompilerParams(
        use_tc_tiling_on_sc=True, needs_layout_passes=False
    ),
)
def _swap01_sc(x_hbm, y_hbm, buf):
    wid = lax.axis_index("s") * NC + lax.axis_index("c")
    lanes = lax.iota(jnp.int32, L)
    col0 = jnp.zeros((L,), jnp.int32)
    col1 = col0 + 1
    for ch in range(NCHUNK):
        base = wid * RPW + ch * CHUNK
        pltpu.sync_copy(x_hbm.at[pl.ds(base, CHUNK)], buf)
        for g in range(GROUPS):
            rows = lanes + (g * L)
            v0 = plsc.load_gather(buf, [rows, col0])
            v1 = plsc.load_gather(buf, [rows, col1])
            plsc.store_scatter(buf, [rows, col0], v1)
            plsc.store_scatter(buf, [rows, col1], v0)
        pltpu.sync_copy(buf, y_hbm.at[pl.ds(base, CHUNK)])


_BM = 512


def _fanout_body(x_ref, o0_ref, o2_ref, o3_ref, o4_ref):
    vt = x_ref[...].T
    o0_ref[...] = vt
    o2_ref[...] = vt
    o4_ref[...] = vt
    o3_ref[...] = jnp.concatenate([vt[1:2, :], vt[0:1, :], vt[2:, :]], axis=0)


_fanout_tc = pl.pallas_call(
    _fanout_body,
    grid=(B // _BM,),
    in_specs=[pl.BlockSpec((_BM, T), lambda i: (i, 0))],
    out_specs=[pl.BlockSpec((T, _BM), lambda i: (0, i)) for _ in range(4)],
    out_shape=[jax.ShapeDtypeStruct((T, B), jnp.float32) for _ in range(4)],
)


def kernel(x):
    y = _swap01_sc(x)
    o0, o2, o3, o4 = _fanout_tc(x)
    return (o0.T, y, o2.T, o3.T, o4.T)


# speedup vs baseline: 1.8401x; 1.8401x over previous
import functools

import jax
import jax.numpy as jnp
from jax import lax
from jax.experimental import pallas as pl
from jax.experimental.pallas import tpu as pltpu
from jax.experimental.pallas import tpu_sc as plsc

B, T = 16384, 200
NC, NS, L = 2, 16, 16
NW = NC * NS
CPW = B // NW
SWAP_GROUPS = CPW // L

_OUT_T = jax.ShapeDtypeStruct((T, B), jnp.float32)


@functools.partial(
    pl.kernel,
    out_type=(_OUT_T, _OUT_T),
    mesh=plsc.VectorSubcoreMesh(core_axis_name="c", subcore_axis_name="s"),
    scratch_types=[pltpu.VMEM((T, CPW), jnp.float32)],
    compiler_params=pltpu.CompilerParams(
        use_tc_tiling_on_sc=True, needs_layout_passes=False
    ),
)
def _swap_sc(xt_hbm, o1_hbm, o3_hbm, buf):
    wid = lax.axis_index("s") * NC + lax.axis_index("c")
    cols = pl.ds(wid * CPW, CPW)
    pltpu.sync_copy(xt_hbm.at[:, cols], buf)
    lanes = lax.iota(jnp.int32, L)
    row0 = jnp.zeros((L,), jnp.int32)
    row1 = row0 + 1
    for g in range(SWAP_GROUPS):
        c = lanes + (g * L)
        v0 = plsc.load_gather(buf, [row0, c])
        v1 = plsc.load_gather(buf, [row1, c])
        plsc.store_scatter(buf, [row0, c], v1)
        plsc.store_scatter(buf, [row1, c], v0)
    pltpu.sync_copy(buf, o1_hbm.at[:, cols])
    pltpu.sync_copy(buf, o3_hbm.at[:, cols])


_BN = 1024


def _fanout_body(xt_ref, o0_ref, o2_ref, o4_ref):
    v = xt_ref[...]
    o0_ref[...] = v
    o2_ref[...] = v
    o4_ref[...] = v


_fanout_tc = pl.pallas_call(
    _fanout_body,
    grid=(B // _BN,),
    in_specs=[pl.BlockSpec((T, _BN), lambda i: (0, i))],
    out_specs=[pl.BlockSpec((T, _BN), lambda i: (0, i)) for _ in range(3)],
    out_shape=[_OUT_T for _ in range(3)],
)


def kernel(x):
    xt = x.T
    o1, o3 = _swap_sc(xt)
    o0, o2, o4 = _fanout_tc(xt)
    return (o0.T, o1.T, o2.T, o3.T, o4.T)


# speedup vs baseline: 1.8903x; 1.0273x over previous
import functools

import jax
import jax.numpy as jnp
from jax import lax
from jax.experimental import pallas as pl
from jax.experimental.pallas import tpu as pltpu
from jax.experimental.pallas import tpu_sc as plsc

B, T = 16384, 200
NC, NS, L = 2, 16, 16
NW = NC * NS
CPW = B // NW
SWAP_GROUPS = CPW // L

_OUT_T = jax.ShapeDtypeStruct((T, B), jnp.float32)


@functools.partial(
    pl.kernel,
    out_type=(_OUT_T, _OUT_T),
    mesh=plsc.VectorSubcoreMesh(core_axis_name="c", subcore_axis_name="s"),
    scratch_types=[pltpu.VMEM((T, CPW), jnp.float32)],
    compiler_params=pltpu.CompilerParams(
        use_tc_tiling_on_sc=True, needs_layout_passes=False
    ),
)
def _swap_sc(xt_hbm, o1_hbm, o3_hbm, buf):
    wid = lax.axis_index("s") * NC + lax.axis_index("c")
    cols = pl.ds(wid * CPW, CPW)
    pltpu.sync_copy(xt_hbm.at[:, cols], buf)
    lanes = lax.iota(jnp.int32, L)
    row0 = jnp.zeros((L,), jnp.int32)
    row1 = row0 + 1
    for g in range(SWAP_GROUPS):
        c = lanes + (g * L)
        v0 = plsc.load_gather(buf, [row0, c])
        v1 = plsc.load_gather(buf, [row1, c])
        plsc.store_scatter(buf, [row0, c], v1)
        plsc.store_scatter(buf, [row1, c], v0)
    pltpu.sync_copy(buf, o1_hbm.at[:, cols])
    pltpu.sync_copy(buf, o3_hbm.at[:, cols])


_BN = 2048


def _fanout_body(xt_ref, o0_ref, o2_ref, o4_ref):
    v = xt_ref[...]
    o0_ref[...] = v
    o2_ref[...] = v
    o4_ref[...] = v


_fanout_tc = pl.pallas_call(
    _fanout_body,
    grid=(B // _BN,),
    in_specs=[pl.BlockSpec((T, _BN), lambda i: (0, i))],
    out_specs=[pl.BlockSpec((T, _BN), lambda i: (0, i)) for _ in range(3)],
    out_shape=[_OUT_T for _ in range(3)],
)


def kernel(x):
    xt = x.T
    o1, o3 = _swap_sc(xt)
    o0, o2, o4 = _fanout_tc(xt)
    return (o0.T, o1.T, o2.T, o3.T, o4.T)
